# baseline (speedup 1.0000x reference)
import functools

import jax
import jax.numpy as jnp
from jax import lax
from jax.experimental import pallas as pl
from jax.experimental.pallas import tpu as pltpu
from jax.experimental.pallas import tpu_sc as plsc

_NC = 2
_NS = 16
_NW = _NC * _NS
_C = 8


def _sc_body1(S, C, b0, x_hbm, pe_hbm, out_hbm,
              xb0, xb1, pb0, pb1, sx0, sx1, spe0, spe1, so0, so1):
    D = x_hbm.shape[1]
    SEG = D // 16
    SW = S // _NW
    NCH = SW // C

    xb = (xb0, xb1)
    pb = (pb0, pb1)
    sx = (sx0, sx1)
    spe = (spe0, spe1)
    so = (so0, so1)

    wid = lax.axis_index("s") * _NC + lax.axis_index("c")
    seq_base = wid * SW

    def start_chunk(c, g):
        row = seq_base + c * C
        pltpu.async_copy(pe_hbm.at[pl.ds(row, C)], pb[g], spe[g])
        pltpu.async_copy(x_hbm.at[pl.ds(b0 * S + row, C)], xb[g], sx[g])

    def wait_in(g):
        pltpu.make_async_copy(pe_hbm.at[pl.ds(0, C)], pb[g], spe[g]).wait()
        pltpu.make_async_copy(x_hbm.at[pl.ds(0, C)], xb[g], sx[g]).wait()

    def wait_out(g):
        pltpu.make_async_copy(xb[g], out_hbm.at[pl.ds(0, C)], so[g]).wait()

    def compute(g):
        def row_body(i, _):
            for j in range(SEG):
                sl = pl.ds(j * 16, 16)
                xb[g][i, sl] = xb[g][i, sl] + pb[g][i, sl]
            return 0

        lax.fori_loop(0, C, row_body, 0)

    def store_chunk(c, g):
        row = seq_base + c * C
        pltpu.async_copy(xb[g], out_hbm.at[pl.ds(row, C)], so[g])

    start_chunk(0, 0)

    def kblock(k, _):
        for pc in (0, 1):
            c = 2 * k + pc
            g, og = pc, 1 - pc

            @pl.when(c + 1 < NCH)
            def _prefetch():
                @pl.when(c >= 1)
                def _drain():
                    wait_out(og)

                start_chunk(c + 1, og)

            wait_in(g)
            compute(g)
            store_chunk(c, g)
        return 0

    lax.fori_loop(0, NCH // 2, kblock, 0)
    wait_out(0)
    wait_out(1)


def _sc_last_batch(x2, pe, S, D, b0):
    mesh = plsc.VectorSubcoreMesh(core_axis_name="c", subcore_axis_name="s")
    return pl.kernel(
        functools.partial(_sc_body1, S, _C, b0),
        out_type=jax.ShapeDtypeStruct((S, D), x2.dtype),
        mesh=mesh,
        scratch_types=(
            [pltpu.VMEM((_C, D), jnp.float32) for _ in range(4)]
            + [pltpu.SemaphoreType.DMA for _ in range(6)]
        ),
    )(x2, pe[:S])


def _tc_add_kernel(x_ref, pe_ref, o_ref):
    o_ref[...] = x_ref[...] + pe_ref[...]


def _tc_batches(x, pe, nb):
    B, S, D = x.shape
    BS = 512
    grid = (S // BS, nb)
    return pl.pallas_call(
        _tc_add_kernel,
        grid=grid,
        in_specs=[
            pl.BlockSpec((1, BS, D), lambda i, j: (j, i, 0)),
            pl.BlockSpec((BS, D), lambda i, j: (i, 0)),
        ],
        out_specs=pl.BlockSpec((1, BS, D), lambda i, j: (j, i, 0)),
        out_shape=jax.ShapeDtypeStruct((nb, S, D), x.dtype),
    )(x, pe[:S])


def kernel(x, pe):
    B, S, D = x.shape
    out_tc = _tc_batches(x, pe, B - 1)
    out_sc = _sc_last_batch(x.reshape(B * S, D), pe, S, D, B - 1)
    return jnp.concatenate([out_tc, out_sc[None]], axis=0)


# speedup vs baseline: 1.8081x; 1.8081x over previous
import jax
import jax.numpy as jnp
from jax.experimental import pallas as pl


def _tc_add_kernel(x_ref, pe_ref, o_ref):
    o_ref[...] = x_ref[...] + pe_ref[...]


def kernel(x, pe):
    B, S, D = x.shape
    BS = 512
    nper = S // BS
    x2 = x.reshape(B * S, D)
    out = pl.pallas_call(
        _tc_add_kernel,
        grid=(B * S // BS,),
        in_specs=[
            pl.BlockSpec((BS, D), lambda i: (i, 0)),
            pl.BlockSpec((BS, D), lambda i: (i % nper, 0)),
        ],
        out_specs=pl.BlockSpec((BS, D), lambda i: (i, 0)),
        out_shape=jax.ShapeDtypeStruct((B * S, D), x.dtype),
    )(x2, pe[:S])
    return out.reshape(B, S, D)


# speedup vs baseline: 2.0928x; 1.1575x over previous
import jax
import jax.numpy as jnp
from jax.experimental import pallas as pl


def _tc_add_kernel(x_ref, pe_ref, o_ref):
    o_ref[...] = x_ref[...] + pe_ref[...]


def kernel(x, pe):
    B, S, D = x.shape
    BS = 512
    nper = S // BS
    x2 = x.reshape(B * S, D)
    out = pl.pallas_call(
        _tc_add_kernel,
        grid=(nper, B),
        in_specs=[
            pl.BlockSpec((BS, D), lambda i, j: (j * nper + i, 0)),
            pl.BlockSpec((BS, D), lambda i, j: (i, 0)),
        ],
        out_specs=pl.BlockSpec((BS, D), lambda i, j: (j * nper + i, 0)),
        out_shape=jax.ShapeDtypeStruct((B * S, D), x.dtype),
    )(x2, pe[:S])
    return out.reshape(B, S, D)


# speedup vs baseline: 2.2056x; 1.0539x over previous
import functools

import jax
import jax.numpy as jnp
from jax.experimental import pallas as pl


def _tc_add_kernel(nper, BS, x_ref, pe_ref, o_ref):
    i = pl.program_id(0)
    off = (i % nper) * BS
    o_ref[...] = x_ref[...] + pe_ref[pl.ds(off, BS), :]


def kernel(x, pe):
    B, S, D = x.shape
    BS = 512
    nper = S // BS
    x2 = x.reshape(B * S, D)
    out = pl.pallas_call(
        functools.partial(_tc_add_kernel, nper, BS),
        grid=(B * S // BS,),
        in_specs=[
            pl.BlockSpec((BS, D), lambda i: (i, 0)),
            pl.BlockSpec((S, D), lambda i: (0, 0)),
        ],
        out_specs=pl.BlockSpec((BS, D), lambda i: (i, 0)),
        out_shape=jax.ShapeDtypeStruct((B * S, D), x.dtype),
    )(x2, pe[:S])
    return out.reshape(B, S, D)


# speedup vs baseline: 2.4009x; 1.0886x over previous
import functools

import jax
import jax.numpy as jnp
from jax.experimental import pallas as pl


def _tc_add_kernel(nper, BS, x_ref, pe_ref, o_ref):
    i = pl.program_id(0)
    off = (i % nper) * BS
    o_ref[...] = x_ref[...] + pe_ref[pl.ds(off, BS), :]


def kernel(x, pe):
    B, S, D = x.shape
    BS = 1024
    nper = S // BS
    x2 = x.reshape(B * S, D)
    out = pl.pallas_call(
        functools.partial(_tc_add_kernel, nper, BS),
        grid=(B * S // BS,),
        in_specs=[
            pl.BlockSpec((BS, D), lambda i: (i, 0)),
            pl.BlockSpec((S, D), lambda i: (0, 0)),
        ],
        out_specs=pl.BlockSpec((BS, D), lambda i: (i, 0)),
        out_shape=jax.ShapeDtypeStruct((B * S, D), x.dtype),
    )(x2, pe[:S])
    return out.reshape(B, S, D)


# speedup vs baseline: 2.4741x; 1.0305x over previous
import functools

import jax
import jax.numpy as jnp
from jax.experimental import pallas as pl


def _tc_add_kernel(nper, BS, x_ref, pe_ref, o_ref):
    i = pl.program_id(0)
    off = (i % nper) * BS
    o_ref[...] = x_ref[...] + pe_ref[pl.ds(off, BS), :]


def kernel(x, pe):
    B, S, D = x.shape
    BS = 2048
    nper = S // BS
    x2 = x.reshape(B * S, D)
    out = pl.pallas_call(
        functools.partial(_tc_add_kernel, nper, BS),
        grid=(B * S // BS,),
        in_specs=[
            pl.BlockSpec((BS, D), lambda i: (i, 0)),
            pl.BlockSpec((S, D), lambda i: (0, 0)),
        ],
        out_specs=pl.BlockSpec((BS, D), lambda i: (i, 0)),
        out_shape=jax.ShapeDtypeStruct((B * S, D), x.dtype),
    )(x2, pe[:S])
    return out.reshape(B, S, D)
